# baseline (speedup 1.0000x reference)
import functools

import jax
import jax.numpy as jnp
from jax.experimental import pallas as pl
from jax.experimental.pallas import tpu as pltpu

B = 1
N = 50000
T = 24
F = 10
H = 64
HEADS = 4
HOR = 6
E = 800000


def _gru_cell(xt, h, Wih, Whh, bih, bhh):
    gi = xt @ Wih.T + bih
    gh = h @ Whh.T + bhh
    i_r, i_z, i_n = jnp.split(gi, 3, axis=-1)
    h_r, h_z, h_n = jnp.split(gh, 3, axis=-1)
    r = jax.nn.sigmoid(i_r + h_r)
    z = jax.nn.sigmoid(i_z + h_z)
    n = jnp.tanh(i_n + r * h_n)
    return (1.0 - z) * n + z * h


def _gat(h, src, dst, M, W, asrc, adst, b):
    x = (h @ W).reshape(M, HEADS, H)
    a_s = (x * asrc[None, :, :]).sum(-1)
    a_d = (x * adst[None, :, :]).sum(-1)
    loops = jnp.arange(M, dtype=src.dtype)
    s_idx = jnp.concatenate([src, loops])
    d_idx = jnp.concatenate([dst, loops])
    e = jax.nn.leaky_relu(a_s[s_idx] + a_d[d_idx], 0.2)
    emax = jax.lax.stop_gradient(jax.ops.segment_max(e, d_idx, num_segments=M))
    ee = jnp.exp(e - emax[d_idx])
    denom = jax.ops.segment_sum(ee, d_idx, num_segments=M)
    msg = jax.ops.segment_sum(ee[:, :, None] * x[s_idx], d_idx, num_segments=M)
    out = msg / (denom[:, :, None] + 1e-16)
    return out.mean(axis=1) + b


def _ln(x, g, b):
    m = x.mean(-1, keepdims=True)
    v = ((x - m) ** 2).mean(-1, keepdims=True)
    return (x - m) / jnp.sqrt(v + 1e-5) * g + b


def _fc_body(h_ref, w_ref, b_ref, o_ref):
    o_ref[...] = jnp.dot(h_ref[...], w_ref[...],
                         preferred_element_type=jnp.float32) + b_ref[...]


def _fc(h, fc_W, fc_b):
    M = h.shape[0]
    TM = 1000
    grid = (M // TM,)
    return pl.pallas_call(
        _fc_body,
        grid=grid,
        in_specs=[
            pl.BlockSpec((TM, H), lambda i: (i, 0)),
            pl.BlockSpec((H, HOR), lambda i: (0, 0)),
            pl.BlockSpec((1, HOR), lambda i: (0, 0)),
        ],
        out_specs=pl.BlockSpec((TM, HOR), lambda i: (i, 0)),
        out_shape=jax.ShapeDtypeStruct((M, HOR), jnp.float32),
    )(h, fc_W, fc_b.reshape(1, HOR))


def kernel(x, edge_index, gru_Wih0, gru_Whh0, gru_bih0, gru_bhh0, gru_Wih1, gru_Whh1, gru_bih1, gru_bhh1, gat1_W, gat1_asrc, gat1_adst, gat1_b, gat2_W, gat2_asrc, gat2_adst, gat2_b, ln1_g, ln1_b, ln2_g, ln2_b, fc_W, fc_b):
    Bq, Nq, Tq, Fq = x.shape
    M = Bq * Nq
    xs = jnp.swapaxes(x.reshape(M, Tq, Fq), 0, 1)

    def step(carry, xt):
        h0, h1 = carry
        h0 = _gru_cell(xt, h0, gru_Wih0, gru_Whh0, gru_bih0, gru_bhh0)
        h1 = _gru_cell(h0, h1, gru_Wih1, gru_Whh1, gru_bih1, gru_bhh1)
        return (h0, h1), None

    init = (jnp.zeros((M, H), jnp.float32), jnp.zeros((M, H), jnp.float32))
    (h0, h1), _ = jax.lax.scan(step, init, xs)
    h = h1
    offsets = (jnp.arange(Bq, dtype=edge_index.dtype) * Nq)[:, None, None]
    bei = (edge_index[None, :, :] + offsets).reshape(2, -1)
    src, dst = bei[0], bei[1]
    g = _gat(h, src, dst, M, gat1_W, gat1_asrc, gat1_adst, gat1_b)
    h = _ln(jax.nn.relu(g) + h, ln1_g, ln1_b)
    g = _gat(h, src, dst, M, gat2_W, gat2_asrc, gat2_adst, gat2_b)
    h = _ln(jax.nn.relu(g) + h, ln2_g, ln2_b)
    return _fc(h, fc_W, fc_b).reshape(Bq, Nq, HOR)


# speedup vs baseline: 15.5455x; 15.5455x over previous
import functools

import jax
import jax.numpy as jnp
from jax import lax
from jax.experimental import pallas as pl
from jax.experimental.pallas import tpu as pltpu
from jax.experimental.pallas import tpu_sc as plsc

N = 50000
T = 24
F = 10
H = 64
HEADS = 4
HOR = 6
E = 800000

S = 4096
CPS = 7
MP = 2 * CPS * S
NSUB = 16
ROWS_PER_TILE = S // NSUB
EPT = E // NSUB
BLK = 2000
NBLK = EPT // BLK
GROUPS = BLK // 16
BW = 64
SEL = 2048


def _gru_body(x_ref, wih0, whh0, bih0, bhh0, wih1, whh1, bih1, bhh1, h_ref):
    tm = x_ref.shape[0]
    h0 = jnp.zeros((tm, H), jnp.float32)
    h1 = jnp.zeros((tm, H), jnp.float32)
    for t in range(T):
        xt = x_ref[:, t, :]
        gi = jnp.dot(xt, wih0[...], preferred_element_type=jnp.float32) + bih0[...]
        gh = jnp.dot(h0, whh0[...], preferred_element_type=jnp.float32) + bhh0[...]
        r = jax.nn.sigmoid(gi[:, 0:H] + gh[:, 0:H])
        z = jax.nn.sigmoid(gi[:, H:2 * H] + gh[:, H:2 * H])
        n = jnp.tanh(gi[:, 2 * H:3 * H] + r * gh[:, 2 * H:3 * H])
        h0 = (1.0 - z) * n + z * h0
        gi = jnp.dot(h0, wih1[...], preferred_element_type=jnp.float32) + bih1[...]
        gh = jnp.dot(h1, whh1[...], preferred_element_type=jnp.float32) + bhh1[...]
        r = jax.nn.sigmoid(gi[:, 0:H] + gh[:, 0:H])
        z = jax.nn.sigmoid(gi[:, H:2 * H] + gh[:, H:2 * H])
        n = jnp.tanh(gi[:, 2 * H:3 * H] + r * gh[:, 2 * H:3 * H])
        h1 = (1.0 - z) * n + z * h1
    h_ref[...] = h1


def _gru(x3, wih0, whh0, bih0, bhh0, wih1, whh1, bih1, bhh1):
    m = x3.shape[0]
    tm = 1000
    grid = (m // tm,)
    rep = lambda shp: pl.BlockSpec(shp, lambda i: tuple(0 for _ in shp))
    return pl.pallas_call(
        _gru_body,
        grid=grid,
        in_specs=[
            pl.BlockSpec((tm, T, F), lambda i: (i, 0, 0)),
            rep((F, 3 * H)), rep((H, 3 * H)), rep((1, 3 * H)), rep((1, 3 * H)),
            rep((H, 3 * H)), rep((H, 3 * H)), rep((1, 3 * H)), rep((1, 3 * H)),
        ],
        out_specs=pl.BlockSpec((tm, H), lambda i: (i, 0)),
        out_shape=jax.ShapeDtypeStruct((m, H), jnp.float32),
    )(x3, wih0, whh0, bih0.reshape(1, -1), bhh0.reshape(1, -1),
      wih1, whh1, bih1.reshape(1, -1), bhh1.reshape(1, -1))


def _gat_dense_body(h_ref, w_ref, am_ref, dm_ref, x4_ref, as_ref, ad_ref):
    x4 = jnp.dot(h_ref[...], w_ref[...], preferred_element_type=jnp.float32)
    x4_ref[...] = x4
    as_ref[...] = jnp.dot(x4, am_ref[...], preferred_element_type=jnp.float32)
    ad_ref[...] = jnp.dot(x4, dm_ref[...], preferred_element_type=jnp.float32)


def _gat_dense(h, W, asrc_mat, adst_mat):
    m = h.shape[0]
    tm = 1000
    grid = (m // tm,)
    rep = lambda shp: pl.BlockSpec(shp, lambda i: tuple(0 for _ in shp))
    return pl.pallas_call(
        _gat_dense_body,
        grid=grid,
        in_specs=[
            pl.BlockSpec((tm, H), lambda i: (i, 0)),
            rep((H, HEADS * H)), rep((HEADS * H, 16)), rep((HEADS * H, 16)),
        ],
        out_specs=[
            pl.BlockSpec((tm, HEADS * H), lambda i: (i, 0)),
            pl.BlockSpec((tm, 16), lambda i: (i, 0)),
            pl.BlockSpec((tm, 16), lambda i: (i, 0)),
        ],
        out_shape=[
            jax.ShapeDtypeStruct((m, HEADS * H), jnp.float32),
            jax.ShapeDtypeStruct((m, 16), jnp.float32),
            jax.ShapeDtypeStruct((m, 16), jnp.float32),
        ],
    )(h, W, asrc_mat, adst_mat)


def _edge_sc_body(src_hbm, dst_hbm, asrc_hbm, adst_hbm, x4_hbm,
                  msg_out, den_out,
                  eblk_s, eblk_d, sel_s, sel_dl,
                  idx_s, idx_dg, idx_dl,
                  as_rows, ad_rows, ee_rows, x4_rows, sc_rows,
                  msg_sh, den_sh):
    cid = lax.axis_index("c")
    sid = lax.axis_index("s")
    i16z = jnp.zeros((16,), jnp.int32)
    f16z = jnp.zeros((16,), jnp.float32)

    def zsel(i, _):
        sel_s[pl.ds(i * 16, 16)] = i16z
        sel_dl[pl.ds(i * 16, 16)] = i16z
        return 0
    lax.fori_loop(0, SEL // 16, zsel, 0)

    def chunk_body(cc, _):
        c = cid * CPS + cc
        base = c * S
        def zrow(i, _):
            for j in range(16):
                sc_rows[i, pl.ds(j * 16, 16)] = f16z
            ee_rows[i, :] = f16z
            return 0
        lax.fori_loop(0, BW, zrow, 0)
        for q in range(ROWS_PER_TILE // BW):
            r0 = sid * ROWS_PER_TILE + q * BW
            pltpu.sync_copy(sc_rows, msg_sh.at[pl.ds(r0, BW)])
            pltpu.sync_copy(ee_rows, den_sh.at[pl.ds(r0, BW)])
        plsc.subcore_barrier()

        def block_body(blk, _):
            eoff = sid * EPT + blk * BLK
            pltpu.sync_copy(src_hbm.at[pl.ds(eoff, BLK)], eblk_s)
            pltpu.sync_copy(dst_hbm.at[pl.ds(eoff, BLK)], eblk_d)

            def scan_body(g, cnt):
                d16 = eblk_d[pl.ds(g * 16, 16)]
                s16 = eblk_s[pl.ds(g * 16, 16)]
                inr = (d16 >= base) & (d16 < base + S)
                pos = plsc.cumsum(inr.astype(jnp.int32))
                idx = jnp.maximum(pos + (cnt - 1), 0)
                plsc.store_scatter(sel_s, [idx], s16, mask=inr)
                plsc.store_scatter(sel_dl, [idx], d16 - base, mask=inr)
                return cnt + jnp.sum(inr.astype(jnp.int32))
            cnt = lax.fori_loop(0, GROUPS, scan_body, jnp.int32(0))
            nb = (cnt + BW - 1) // BW

            def batch_body(b, _):
                for j in range(BW // 16):
                    off = b * BW + j * 16
                    sv = sel_s[pl.ds(off, 16)]
                    dlv = sel_dl[pl.ds(off, 16)]
                    idx_s[pl.ds(j * 16, 16)] = sv
                    idx_dl[pl.ds(j * 16, 16)] = dlv
                    idx_dg[pl.ds(j * 16, 16)] = dlv + base
                pltpu.sync_copy(asrc_hbm.at[idx_s], as_rows)
                pltpu.sync_copy(adst_hbm.at[idx_dg], ad_rows)
                pltpu.sync_copy(x4_hbm.at[idx_s], x4_rows)

                def edge_body(i, _):
                    a = as_rows[i, :] + ad_rows[i, :]
                    e = jnp.where(a >= 0.0, a, a * 0.2)
                    ee = jnp.exp(e)
                    pos = jnp.full((16,), b * BW, jnp.int32) + i
                    ee = jnp.where(pos < cnt, ee, 0.0)
                    ee_rows[i, :] = ee
                    for k in range(HEADS):
                        sk = plsc.load_gather(
                            ee_rows,
                            [jnp.full((16,), i, jnp.int32),
                             jnp.full((16,), k, jnp.int32)])
                        for j in range(H // 16):
                            col = k * H + j * 16
                            sc_rows[i, pl.ds(col, 16)] = (
                                x4_rows[i, pl.ds(col, 16)] * sk)
                    return 0
                lax.fori_loop(0, BW, edge_body, 0)
                pltpu.sync_copy(ee_rows, den_sh.at[idx_dl], add=True)
                pltpu.sync_copy(sc_rows, msg_sh.at[idx_dl], add=True)
                return 0
            lax.fori_loop(0, nb, batch_body, 0)
            return 0
        lax.fori_loop(0, NBLK, block_body, 0)
        plsc.subcore_barrier()
        r0 = sid * ROWS_PER_TILE
        pltpu.sync_copy(msg_sh.at[pl.ds(r0, ROWS_PER_TILE)],
                        msg_out.at[pl.ds(base + r0, ROWS_PER_TILE)])
        pltpu.sync_copy(den_sh.at[pl.ds(r0, ROWS_PER_TILE)],
                        den_out.at[pl.ds(base + r0, ROWS_PER_TILE)])
        plsc.subcore_barrier()
        return 0
    lax.fori_loop(0, CPS, chunk_body, 0)


def _edge_sc(src, dst, a_s, a_d, x4):
    mesh = plsc.VectorSubcoreMesh(core_axis_name="c", subcore_axis_name="s")
    f = pl.kernel(
        _edge_sc_body,
        compiler_params=pltpu.CompilerParams(
            use_tc_tiling_on_sc=False, needs_layout_passes=False),
        out_type=[
            jax.ShapeDtypeStruct((MP, HEADS * H), jnp.float32),
            jax.ShapeDtypeStruct((MP, 16), jnp.float32),
        ],
        mesh=mesh,
        scratch_types=[
            pltpu.VMEM((BLK,), jnp.int32),
            pltpu.VMEM((BLK,), jnp.int32),
            pltpu.VMEM((SEL,), jnp.int32),
            pltpu.VMEM((SEL,), jnp.int32),
            pltpu.VMEM((BW,), jnp.int32),
            pltpu.VMEM((BW,), jnp.int32),
            pltpu.VMEM((BW,), jnp.int32),
            pltpu.VMEM((BW, 16), jnp.float32),
            pltpu.VMEM((BW, 16), jnp.float32),
            pltpu.VMEM((BW, 16), jnp.float32),
            pltpu.VMEM((BW, HEADS * H), jnp.float32),
            pltpu.VMEM((BW, HEADS * H), jnp.float32),
            pltpu.VMEM_SHARED((S, HEADS * H), jnp.float32),
            pltpu.VMEM_SHARED((S, 16), jnp.float32),
        ],
    )
    return f(src, dst, a_s, a_d, x4)


def _finalize_body(msg_ref, den_ref, x4_ref, as_ref, ad_ref, h_ref,
                   exp_ref, mean_ref, gb_ref, lng_ref, lnb_ref, out_ref):
    a = as_ref[...] + ad_ref[...]
    e = jnp.where(a >= 0.0, a, a * 0.2)
    ee_self = jnp.exp(e)
    ee_exp = jnp.dot(ee_self, exp_ref[...], preferred_element_type=jnp.float32)
    msgf = msg_ref[...] + ee_exp * x4_ref[...]
    denf = jnp.dot(den_ref[...] + ee_self, exp_ref[...],
                   preferred_element_type=jnp.float32)
    ratio = msgf / (denf + 1e-16)
    g = jnp.dot(ratio, mean_ref[...], preferred_element_type=jnp.float32)
    g = g + gb_ref[...]
    r = jnp.maximum(g, 0.0) + h_ref[...]
    mu = jnp.mean(r, axis=-1, keepdims=True)
    var = jnp.mean((r - mu) ** 2, axis=-1, keepdims=True)
    out_ref[...] = (r - mu) / jnp.sqrt(var + 1e-5) * lng_ref[...] + lnb_ref[...]


def _finalize(msg, den, x4, a_s, a_d, h, expand, meanmat, gb, lng, lnb):
    m = h.shape[0]
    tm = 1000
    grid = (m // tm,)
    rep = lambda shp: pl.BlockSpec(shp, lambda i: tuple(0 for _ in shp))
    return pl.pallas_call(
        _finalize_body,
        grid=grid,
        in_specs=[
            pl.BlockSpec((tm, HEADS * H), lambda i: (i, 0)),
            pl.BlockSpec((tm, 16), lambda i: (i, 0)),
            pl.BlockSpec((tm, HEADS * H), lambda i: (i, 0)),
            pl.BlockSpec((tm, 16), lambda i: (i, 0)),
            pl.BlockSpec((tm, 16), lambda i: (i, 0)),
            pl.BlockSpec((tm, H), lambda i: (i, 0)),
            rep((16, HEADS * H)), rep((HEADS * H, H)),
            rep((1, H)), rep((1, H)), rep((1, H)),
        ],
        out_specs=pl.BlockSpec((tm, H), lambda i: (i, 0)),
        out_shape=jax.ShapeDtypeStruct((m, H), jnp.float32),
    )(msg, den, x4, a_s, a_d, h, expand, meanmat,
      gb.reshape(1, -1), lng.reshape(1, -1), lnb.reshape(1, -1))


def _fc_body(h_ref, w_ref, b_ref, o_ref):
    o_ref[...] = jnp.dot(h_ref[...], w_ref[...],
                         preferred_element_type=jnp.float32) + b_ref[...]


def _fc(h, fc_W, fc_b):
    m = h.shape[0]
    tm = 1000
    grid = (m // tm,)
    return pl.pallas_call(
        _fc_body,
        grid=grid,
        in_specs=[
            pl.BlockSpec((tm, H), lambda i: (i, 0)),
            pl.BlockSpec((H, HOR), lambda i: (0, 0)),
            pl.BlockSpec((1, HOR), lambda i: (0, 0)),
        ],
        out_specs=pl.BlockSpec((tm, HOR), lambda i: (i, 0)),
        out_shape=jax.ShapeDtypeStruct((m, HOR), jnp.float32),
    )(h, fc_W, fc_b.reshape(1, HOR))


def _attn_mats(asrc, adst):
    eye = jnp.eye(16, dtype=jnp.float32)[:HEADS]
    am = (asrc[:, :, None] * eye[:, None, :]).reshape(HEADS * H, 16)
    dm = (adst[:, :, None] * eye[:, None, :]).reshape(HEADS * H, 16)
    return am, dm


def _gat_layer(h, src, dst, W, asrc, adst, b, lng, lnb, expand, meanmat):
    am, dm = _attn_mats(asrc, adst)
    x4, a_s, a_d = _gat_dense(h, W, am, dm)
    a_dp = jnp.pad(a_d, ((0, MP - a_d.shape[0]), (0, 0)))
    msg, den = _edge_sc(src, dst, a_s, a_dp, x4)
    m = h.shape[0]
    return _finalize(msg[:m], den[:m], x4, a_s, a_d, h,
                     expand, meanmat, b, lng, lnb)


def kernel(x, edge_index, gru_Wih0, gru_Whh0, gru_bih0, gru_bhh0, gru_Wih1, gru_Whh1, gru_bih1, gru_bhh1, gat1_W, gat1_asrc, gat1_adst, gat1_b, gat2_W, gat2_asrc, gat2_adst, gat2_b, ln1_g, ln1_b, ln2_g, ln2_b, fc_W, fc_b):
    Bq, Nq, Tq, Fq = x.shape
    m = Bq * Nq
    x3 = x.reshape(m, Tq, Fq)
    h = _gru(x3, gru_Wih0.T, gru_Whh0.T, gru_bih0, gru_bhh0,
             gru_Wih1.T, gru_Whh1.T, gru_bih1, gru_bhh1)

    src = edge_index[0]
    dst = edge_index[1]
    eye16 = jnp.eye(16, dtype=jnp.float32)[:, :HEADS]
    expand = jnp.kron(eye16, jnp.ones((1, H), jnp.float32))
    meanmat = jnp.kron(jnp.ones((HEADS, 1), jnp.float32),
                       jnp.eye(H, dtype=jnp.float32)) / HEADS

    h = _gat_layer(h, src, dst, gat1_W, gat1_asrc, gat1_adst, gat1_b,
                   ln1_g, ln1_b, expand, meanmat)
    h = _gat_layer(h, src, dst, gat2_W, gat2_asrc, gat2_adst, gat2_b,
                   ln2_g, ln2_b, expand, meanmat)
    return _fc(h, fc_W, fc_b).reshape(Bq, Nq, HOR)
